# baseline (speedup 1.0000x reference)
import jax
import jax.numpy as jnp
from jax import lax
from jax.experimental import pallas as pl
from jax.experimental.pallas import tpu as pltpu
from jax.experimental.pallas import tpu_sc as plsc

N_BUCKETS = 8192
BS = 1048576
L = 16
NW = 32
PER_W = BS // NW
CHUNK = 16384
N_CHUNKS = PER_W // CHUNK
INV_N = 1.0 / N_BUCKETS

TC_ROWS = 1024
TC_BLOCK = 128


def _t_body(ids_ref, u_ref, t_ref):
    t_ref[...] = (ids_ref[...].astype(jnp.float32) + u_ref[...]) * INV_N


def _p_body(logits_hbm, ids_hbm, u_hbm, p_hbm,
            h_v, ids_v0, ids_v1, u_v0, u_v1, p_v0, p_v1,
            sem_h, in_sem0, in_sem1, out_sem0, out_sem1):
    wid = lax.axis_index("s") * 2 + lax.axis_index("c")
    base = wid * PER_W
    ids_bufs = (ids_v0, ids_v1)
    u_bufs = (u_v0, u_v1)
    p_bufs = (p_v0, p_v1)
    in_sems = (in_sem0, in_sem1)
    out_sems = (out_sem0, out_sem1)

    h_cp = pltpu.async_copy(logits_hbm, h_v, sem_h)

    def start_in(ci):
        b = ci % 2
        off = base + ci * CHUNK
        return (pltpu.async_copy(ids_hbm.at[pl.ds(off, CHUNK)], ids_bufs[b], in_sems[b]),
                pltpu.async_copy(u_hbm.at[pl.ds(off, CHUNK)], u_bufs[b], in_sems[b]))

    in_cp = start_in(0)
    h_cp.wait()

    def max_body(i, m_vec):
        return jnp.maximum(m_vec, h_v[pl.ds(i * L, L)])

    m_vec = lax.fori_loop(0, N_BUCKETS // L, max_body,
                          jnp.full((L,), -jnp.inf, jnp.float32), unroll=8)
    m = jnp.max(m_vec)

    def exp_body(i, s_vec):
        sl = pl.ds(i * L, L)
        e = jnp.exp(h_v[sl] - m)
        h_v[sl] = e
        return s_vec + e

    s_vec = lax.fori_loop(0, N_BUCKETS // L, exp_body,
                          jnp.zeros((L,), jnp.float32), unroll=8)
    scale = jnp.full((L,), float(N_BUCKETS), jnp.float32) / jnp.broadcast_to(
        jnp.sum(s_vec), (L,))

    out_cp = [None, None]
    for ci in range(N_CHUNKS):
        b = ci % 2
        next_in = start_in(ci + 1) if ci + 1 < N_CHUNKS else None
        for cp in in_cp:
            cp.wait()
        if out_cp[b] is not None:
            out_cp[b].wait()
        ids_b, u_b, p_b = ids_bufs[b], u_bufs[b], p_bufs[b]

        @plsc.parallel_loop(0, CHUNK, step=L, unroll=8)
        def _compute(i):
            s = ids_b[pl.ds(i, L)].astype(jnp.float32) + u_b[pl.ds(i, L)]
            idx = jnp.minimum(s.astype(jnp.int32), N_BUCKETS - 1)
            p_b[pl.ds(i, L)] = plsc.load_gather(h_v, [idx]) * scale

        off = base + ci * CHUNK
        out_cp[b] = pltpu.async_copy(p_b, p_hbm.at[pl.ds(off, CHUNK)], out_sems[b])
        if next_in is not None:
            in_cp = next_in
    for cp in out_cp:
        if cp is not None:
            cp.wait()


def kernel(logits, ids, u):
    p_call = pl.kernel(
        _p_body,
        out_type=jax.ShapeDtypeStruct((BS,), jnp.float32),
        mesh=plsc.VectorSubcoreMesh(core_axis_name="c", subcore_axis_name="s"),
        compiler_params=pltpu.CompilerParams(needs_layout_passes=False),
        scratch_types=[
            pltpu.VMEM((N_BUCKETS,), jnp.float32),
            pltpu.VMEM((CHUNK,), jnp.int32),
            pltpu.VMEM((CHUNK,), jnp.int32),
            pltpu.VMEM((CHUNK,), jnp.float32),
            pltpu.VMEM((CHUNK,), jnp.float32),
            pltpu.VMEM((CHUNK,), jnp.float32),
            pltpu.VMEM((CHUNK,), jnp.float32),
            pltpu.SemaphoreType.DMA,
            pltpu.SemaphoreType.DMA,
            pltpu.SemaphoreType.DMA,
            pltpu.SemaphoreType.DMA,
            pltpu.SemaphoreType.DMA,
        ],
    )
    p = p_call(logits, ids, u)

    t = pl.pallas_call(
        _t_body,
        grid=(TC_ROWS // TC_BLOCK,),
        in_specs=[pl.BlockSpec((TC_BLOCK, BS // TC_ROWS), lambda i: (i, 0)),
                  pl.BlockSpec((TC_BLOCK, BS // TC_ROWS), lambda i: (i, 0))],
        out_specs=pl.BlockSpec((TC_BLOCK, BS // TC_ROWS), lambda i: (i, 0)),
        out_shape=jax.ShapeDtypeStruct((TC_ROWS, BS // TC_ROWS), jnp.float32),
    )(ids.reshape(TC_ROWS, BS // TC_ROWS), u.reshape(TC_ROWS, BS // TC_ROWS))

    return (t.reshape(BS, 1), p)


# speedup vs baseline: 1.4241x; 1.4241x over previous
import jax
import jax.numpy as jnp
from jax import lax
from jax.experimental import pallas as pl
from jax.experimental.pallas import tpu as pltpu
from jax.experimental.pallas import tpu_sc as plsc

N_BUCKETS = 8192
BS = 1048576
L = 16
NW = 32
PER_W = BS // NW
CHUNK = 8192
N_CHUNKS = PER_W // CHUNK
INV_N = 1.0 / N_BUCKETS


def _sampler_body(logits_hbm, ids_hbm, u_hbm, t_hbm, p_hbm,
                  h_v, ids_v0, ids_v1, u_v0, u_v1, t_v0, t_v1, p_v0, p_v1,
                  sem_h, in_sem0, in_sem1, out_sem0, out_sem1):
    wid = lax.axis_index("s") * 2 + lax.axis_index("c")
    base = wid * PER_W
    ids_bufs = (ids_v0, ids_v1)
    u_bufs = (u_v0, u_v1)
    t_bufs = (t_v0, t_v1)
    p_bufs = (p_v0, p_v1)
    in_sems = (in_sem0, in_sem1)
    out_sems = (out_sem0, out_sem1)

    h_cp = pltpu.async_copy(logits_hbm, h_v, sem_h)

    def start_in(ci):
        b = ci % 2
        off = base + ci * CHUNK
        return (pltpu.async_copy(ids_hbm.at[pl.ds(off, CHUNK)], ids_bufs[b], in_sems[b]),
                pltpu.async_copy(u_hbm.at[pl.ds(off, CHUNK)], u_bufs[b], in_sems[b]))

    in_cp = start_in(0)
    h_cp.wait()

    def max_body(i, m_vec):
        return jnp.maximum(m_vec, h_v[pl.ds(i * L, L)])

    m_vec = lax.fori_loop(0, N_BUCKETS // L, max_body,
                          jnp.full((L,), -jnp.inf, jnp.float32), unroll=8)
    m = jnp.max(m_vec)

    def exp_body(i, s_vec):
        sl = pl.ds(i * L, L)
        e = jnp.exp(h_v[sl] - m)
        h_v[sl] = e
        return s_vec + e

    s_vec = lax.fori_loop(0, N_BUCKETS // L, exp_body,
                          jnp.zeros((L,), jnp.float32), unroll=8)
    scale = jnp.full((L,), float(N_BUCKETS), jnp.float32) / jnp.broadcast_to(
        jnp.sum(s_vec), (L,))

    out_cp = [None, None]
    for ci in range(N_CHUNKS):
        b = ci % 2
        next_in = start_in(ci + 1) if ci + 1 < N_CHUNKS else None
        for cp in in_cp:
            cp.wait()
        if out_cp[b] is not None:
            for cp in out_cp[b]:
                cp.wait()
        ids_b, u_b, t_b, p_b = ids_bufs[b], u_bufs[b], t_bufs[b], p_bufs[b]

        @plsc.parallel_loop(0, CHUNK, step=L, unroll=8)
        def _compute(i):
            s = ids_b[pl.ds(i, L)].astype(jnp.float32) + u_b[pl.ds(i, L)]
            t_b[pl.ds(i, L)] = s * INV_N
            idx = jnp.minimum(s.astype(jnp.int32), N_BUCKETS - 1)
            p_b[pl.ds(i, L)] = plsc.load_gather(h_v, [idx]) * scale

        off = base + ci * CHUNK
        out_cp[b] = (pltpu.async_copy(t_b, t_hbm.at[pl.ds(off, CHUNK)], out_sems[b]),
                     pltpu.async_copy(p_b, p_hbm.at[pl.ds(off, CHUNK)], out_sems[b]))
        if next_in is not None:
            in_cp = next_in
    for pair in out_cp:
        if pair is not None:
            for cp in pair:
                cp.wait()


def kernel(logits, ids, u):
    sampler = pl.kernel(
        _sampler_body,
        out_type=[jax.ShapeDtypeStruct((BS,), jnp.float32),
                  jax.ShapeDtypeStruct((BS,), jnp.float32)],
        mesh=plsc.VectorSubcoreMesh(core_axis_name="c", subcore_axis_name="s"),
        compiler_params=pltpu.CompilerParams(needs_layout_passes=False),
        scratch_types=[
            pltpu.VMEM((N_BUCKETS,), jnp.float32),
            pltpu.VMEM((CHUNK,), jnp.int32),
            pltpu.VMEM((CHUNK,), jnp.int32),
            pltpu.VMEM((CHUNK,), jnp.float32),
            pltpu.VMEM((CHUNK,), jnp.float32),
            pltpu.VMEM((CHUNK,), jnp.float32),
            pltpu.VMEM((CHUNK,), jnp.float32),
            pltpu.VMEM((CHUNK,), jnp.float32),
            pltpu.VMEM((CHUNK,), jnp.float32),
            pltpu.SemaphoreType.DMA,
            pltpu.SemaphoreType.DMA,
            pltpu.SemaphoreType.DMA,
            pltpu.SemaphoreType.DMA,
            pltpu.SemaphoreType.DMA,
        ],
    )
    t, p = sampler(logits, ids, u)
    return (t[:, None], p)
